# Initial kernel scaffold; baseline (speedup 1.0000x reference)
import jax
import jax.numpy as jnp
from jax.experimental import pallas as pl


def kernel(features, edge_index, weight, bias):
    raise NotImplementedError("write your pallas kernel here")


# speedup vs baseline: 7.4776x; 7.4776x over previous
import functools

import jax
import jax.numpy as jnp
from jax import lax
from jax.experimental import pallas as pl
from jax.experimental.pallas import tpu as pltpu
from jax.experimental.pallas import tpu_sc as plsc

N_NODES = 10000
N_FEATS = 128
N_EDGES = 320000

NC = 2
NS = 16
NW = NC * NS
EDGES_PER_TILE = N_EDGES // NW
CHUNK = 125
CHUNKS_PER_TILE = EDGES_PER_TILE // CHUNK
LANES = 16
CP_ROWS = 200
CP_CHUNKS = N_NODES // CP_ROWS
CP_ROUNDS = -(-CP_CHUNKS // NS)

_MESH = plsc.VectorSubcoreMesh(
    core_axis_name="c", subcore_axis_name="s", num_cores=NC, num_subcores=NS)
_SC_PARAMS = pltpu.CompilerParams(needs_layout_passes=False)


def _degrees_body(src_hbm, dst_hbm, out_src, out_dst, idx_v, hist_v):
    c = lax.axis_index("c")
    s = lax.axis_index("s")
    wid = c * NS + s
    base = wid * EDGES_PER_TILE
    ones = jnp.full((LANES,), 1.0, jnp.float32)
    zeros = jnp.zeros((LANES,), jnp.float32)

    def one_endpoint(ep_hbm, out_hbm):
        pltpu.sync_copy(ep_hbm.at[pl.ds(base, EDGES_PER_TILE)], idx_v)

        def zero_step(i, _):
            hist_v[pl.ds(i * LANES, LANES)] = zeros
            return 0

        lax.fori_loop(0, N_NODES // LANES, zero_step, 0)

        def add_step(i, _):
            idx = idx_v[pl.ds(i * LANES, LANES)]
            plsc.addupdate_scatter(hist_v, [idx], ones)
            return 0

        lax.fori_loop(0, EDGES_PER_TILE // LANES, add_step, 0)
        pltpu.sync_copy(hist_v, out_hbm.at[wid])

    one_endpoint(src_hbm, out_src)
    one_endpoint(dst_hbm, out_dst)


def _degrees(src, dst):
    return pl.kernel(
        _degrees_body,
        out_type=(
            jax.ShapeDtypeStruct((NW, N_NODES), jnp.float32),
            jax.ShapeDtypeStruct((NW, N_NODES), jnp.float32),
        ),
        mesh=_MESH,
        scratch_types=[
            pltpu.VMEM((EDGES_PER_TILE,), jnp.int32),
            pltpu.VMEM((N_NODES,), jnp.float32),
        ],
        compiler_params=_SC_PARAMS,
    )(src, dst)


def _norms_body(hs_ref, hd_ref, ns_ref, nd_ref):
    ds = jnp.sum(hs_ref[...], axis=0)
    dd = jnp.sum(hd_ref[...], axis=0)
    ns_ref[...] = lax.rsqrt(jnp.maximum(ds, 1.0))[:, None]
    nd_ref[...] = lax.rsqrt(jnp.maximum(dd, 1.0))[:, None]


def _norms(hist_src, hist_dst):
    return pl.pallas_call(
        _norms_body,
        out_shape=(
            jax.ShapeDtypeStruct((N_NODES, 1), jnp.float32),
            jax.ShapeDtypeStruct((N_NODES, 1), jnp.float32),
        ),
    )(hist_src, hist_dst)


_MM_BLK = 1000


def _matmul_body(x_ref, n1_ref, w_ref, h_ref):
    h_ref[...] = jnp.dot(x_ref[...] * n1_ref[...], w_ref[...],
                         preferred_element_type=jnp.float32)


def _matmul(features, norm_src, weight):
    return pl.pallas_call(
        _matmul_body,
        grid=(N_NODES // _MM_BLK,),
        in_specs=[
            pl.BlockSpec((_MM_BLK, N_FEATS), lambda i: (i, 0)),
            pl.BlockSpec((_MM_BLK, 1), lambda i: (i, 0)),
            pl.BlockSpec((N_FEATS, N_FEATS), lambda i: (0, 0)),
        ],
        out_specs=pl.BlockSpec((_MM_BLK, N_FEATS), lambda i: (i, 0)),
        out_shape=jax.ShapeDtypeStruct((N_NODES, N_FEATS), jnp.float32),
    )(features, norm_src, weight)


def _message_body(h_hbm, src_hbm, dst_hbm, zeros_hbm, out_hbm,
                  srcblk, dstblk, rows, acc_sh, sem):
    c = lax.axis_index("c")
    s = lax.axis_index("s")
    wid = c * NS + s

    for j in range(CP_ROUNDS):
        cid = s + NS * j

        @pl.when(cid < CP_CHUNKS)
        def _():
            pltpu.sync_copy(zeros_hbm, acc_sh.at[pl.ds(cid * CP_ROWS, CP_ROWS)])

    plsc.subcore_barrier()

    pltpu.sync_copy(src_hbm.at[pl.ds(wid * CHUNKS_PER_TILE, CHUNKS_PER_TILE)], srcblk)
    pltpu.sync_copy(dst_hbm.at[pl.ds(wid * CHUNKS_PER_TILE, CHUNKS_PER_TILE)], dstblk)

    def chunk_step(j, _):
        pltpu.async_copy(h_hbm.at[srcblk.at[j]], rows, sem).wait()
        pltpu.sync_copy(rows, acc_sh.at[dstblk.at[j]], add=True)
        return 0

    lax.fori_loop(0, CHUNKS_PER_TILE, chunk_step, 0)
    plsc.subcore_barrier()

    for j in range(CP_ROUNDS):
        cid = s + NS * j

        @pl.when(cid < CP_CHUNKS)
        def _():
            pltpu.sync_copy(acc_sh.at[pl.ds(cid * CP_ROWS, CP_ROWS)],
                            out_hbm.at[c, pl.ds(cid * CP_ROWS, CP_ROWS)])


def _message(h, src2, dst2, zeros):
    return pl.kernel(
        _message_body,
        out_type=jax.ShapeDtypeStruct((NC, N_NODES, N_FEATS), jnp.float32),
        mesh=_MESH,
        scratch_types=[
            pltpu.VMEM((CHUNKS_PER_TILE, CHUNK), jnp.int32),
            pltpu.VMEM((CHUNKS_PER_TILE, CHUNK), jnp.int32),
            pltpu.VMEM((CHUNK, N_FEATS), jnp.float32),
            pltpu.VMEM_SHARED((N_NODES, N_FEATS), jnp.float32),
            pltpu.SemaphoreType.DMA,
        ],
        compiler_params=_SC_PARAMS,
    )(h, src2, dst2, zeros)


def _finalize_body(p_ref, n2_ref, b_ref, out_ref):
    out_ref[...] = (p_ref[0] + p_ref[1]) * n2_ref[...] + b_ref[...]


def _finalize(partials, norm_dst, bias):
    return pl.pallas_call(
        _finalize_body,
        grid=(N_NODES // _MM_BLK,),
        in_specs=[
            pl.BlockSpec((NC, _MM_BLK, N_FEATS), lambda i: (0, i, 0)),
            pl.BlockSpec((_MM_BLK, 1), lambda i: (i, 0)),
            pl.BlockSpec((1, N_FEATS), lambda i: (0, 0)),
        ],
        out_specs=pl.BlockSpec((_MM_BLK, N_FEATS), lambda i: (i, 0)),
        out_shape=jax.ShapeDtypeStruct((N_NODES, N_FEATS), jnp.float32),
    )(partials, norm_dst, bias)


def kernel(features, edge_index, weight, bias):
    ei = edge_index.astype(jnp.int32)
    src = ei[0]
    dst = ei[1]
    hist_src, hist_dst = _degrees(src, dst)
    norm_src, norm_dst = _norms(hist_src, hist_dst)
    h = _matmul(features, norm_src, weight)
    src2 = src.reshape(NW * CHUNKS_PER_TILE, CHUNK)
    dst2 = dst.reshape(NW * CHUNKS_PER_TILE, CHUNK)
    zeros = jnp.zeros((CP_ROWS, N_FEATS), jnp.float32)
    partials = _message(h, src2, dst2, zeros)
    return _finalize(partials, norm_dst, bias.reshape(1, N_FEATS))


# speedup vs baseline: 9.0994x; 1.2169x over previous
import functools

import jax
import jax.numpy as jnp
from jax import lax
from jax.experimental import pallas as pl
from jax.experimental.pallas import tpu as pltpu
from jax.experimental.pallas import tpu_sc as plsc

N_NODES = 10000
N_FEATS = 128
N_EDGES = 320000

NC = 2
NS = 16
NW = NC * NS
EDGES_PER_TILE = N_EDGES // NW
CHUNK = 125
CHUNKS_PER_TILE = EDGES_PER_TILE // CHUNK
IDXG = 8
N_GROUPS = CHUNKS_PER_TILE // IDXG
LANES = 16
CP_ROWS = 200
CP_CHUNKS = N_NODES // CP_ROWS
CP_ROUNDS = -(-CP_CHUNKS // NS)

_MESH = plsc.VectorSubcoreMesh(
    core_axis_name="c", subcore_axis_name="s", num_cores=NC, num_subcores=NS)
_SC_PARAMS = pltpu.CompilerParams(needs_layout_passes=False)


def _degrees_body(src_hbm, dst_hbm, out_src, out_dst, idx_v, hist_v):
    c = lax.axis_index("c")
    s = lax.axis_index("s")
    wid = c * NS + s
    base = wid * EDGES_PER_TILE
    ones = jnp.full((LANES,), 1.0, jnp.float32)
    zeros = jnp.zeros((LANES,), jnp.float32)

    def one_endpoint(ep_hbm, out_hbm):
        pltpu.sync_copy(ep_hbm.at[pl.ds(base, EDGES_PER_TILE)], idx_v)

        def zero_step(i, _):
            hist_v[pl.ds(i * LANES, LANES)] = zeros
            return 0

        lax.fori_loop(0, N_NODES // LANES, zero_step, 0)

        def add_step(i, _):
            idx = idx_v[pl.ds(i * LANES, LANES)]
            plsc.addupdate_scatter(hist_v, [idx], ones)
            return 0

        lax.fori_loop(0, EDGES_PER_TILE // LANES, add_step, 0)
        pltpu.sync_copy(hist_v, out_hbm.at[wid])

    one_endpoint(src_hbm, out_src)
    one_endpoint(dst_hbm, out_dst)


def _degrees(src, dst):
    return pl.kernel(
        _degrees_body,
        out_type=(
            jax.ShapeDtypeStruct((NW, N_NODES), jnp.float32),
            jax.ShapeDtypeStruct((NW, N_NODES), jnp.float32),
        ),
        mesh=_MESH,
        scratch_types=[
            pltpu.VMEM((EDGES_PER_TILE,), jnp.int32),
            pltpu.VMEM((N_NODES,), jnp.float32),
        ],
        compiler_params=_SC_PARAMS,
    )(src, dst)


def _norms_body(hs_ref, hd_ref, ns_ref, nd_ref):
    ds = jnp.sum(hs_ref[...], axis=0)
    dd = jnp.sum(hd_ref[...], axis=0)
    ns_ref[...] = lax.rsqrt(jnp.maximum(ds, 1.0))[:, None]
    nd_ref[...] = lax.rsqrt(jnp.maximum(dd, 1.0))[:, None]


def _norms(hist_src, hist_dst):
    return pl.pallas_call(
        _norms_body,
        out_shape=(
            jax.ShapeDtypeStruct((N_NODES, 1), jnp.float32),
            jax.ShapeDtypeStruct((N_NODES, 1), jnp.float32),
        ),
    )(hist_src, hist_dst)


_MM_BLK = 1000


def _matmul_body(x_ref, n1_ref, w_ref, h_ref):
    h_ref[...] = jnp.dot(x_ref[...] * n1_ref[...], w_ref[...],
                         preferred_element_type=jnp.float32)


def _matmul(features, norm_src, weight):
    return pl.pallas_call(
        _matmul_body,
        grid=(N_NODES // _MM_BLK,),
        in_specs=[
            pl.BlockSpec((_MM_BLK, N_FEATS), lambda i: (i, 0)),
            pl.BlockSpec((_MM_BLK, 1), lambda i: (i, 0)),
            pl.BlockSpec((N_FEATS, N_FEATS), lambda i: (0, 0)),
        ],
        out_specs=pl.BlockSpec((_MM_BLK, N_FEATS), lambda i: (i, 0)),
        out_shape=jax.ShapeDtypeStruct((N_NODES, N_FEATS), jnp.float32),
    )(features, norm_src, weight)


def _message_body(h_hbm, src_hbm, dst_hbm, zeros_hbm, out_hbm,
                  srcblk, dstblk, rows, acc_sh, sem0, sem1):
    c = lax.axis_index("c")
    s = lax.axis_index("s")
    wid = c * NS + s

    for j in range(CP_ROUNDS):
        cid = s + NS * j

        @pl.when(cid < CP_CHUNKS)
        def _():
            pltpu.sync_copy(zeros_hbm, acc_sh.at[pl.ds(cid * CP_ROWS, CP_ROWS)])

    plsc.subcore_barrier()

    sems = (sem0, sem1)

    def group_step(g, _):
        base = wid * CHUNKS_PER_TILE + g * IDXG
        pltpu.sync_copy(src_hbm.at[pl.ds(base, IDXG)], srcblk)
        pltpu.sync_copy(dst_hbm.at[pl.ds(base, IDXG)], dstblk)
        pltpu.async_copy(h_hbm.at[srcblk.at[0]], rows.at[0], sems[0])
        for j in range(IDXG):
            b = j % 2
            if j + 1 < IDXG:
                pltpu.async_copy(h_hbm.at[srcblk.at[j + 1]],
                                 rows.at[1 - b], sems[1 - b])
            pltpu.make_async_copy(h_hbm.at[srcblk.at[j]],
                                  rows.at[b], sems[b]).wait()
            pltpu.sync_copy(rows.at[b], acc_sh.at[dstblk.at[j]], add=True)
        return 0

    lax.fori_loop(0, N_GROUPS, group_step, 0)
    plsc.subcore_barrier()

    for j in range(CP_ROUNDS):
        cid = s + NS * j

        @pl.when(cid < CP_CHUNKS)
        def _():
            pltpu.sync_copy(acc_sh.at[pl.ds(cid * CP_ROWS, CP_ROWS)],
                            out_hbm.at[c, pl.ds(cid * CP_ROWS, CP_ROWS)])


def _message(h, src2, dst2, zeros):
    return pl.kernel(
        _message_body,
        out_type=jax.ShapeDtypeStruct((NC, N_NODES, N_FEATS), jnp.float32),
        mesh=_MESH,
        scratch_types=[
            pltpu.VMEM((IDXG, CHUNK), jnp.int32),
            pltpu.VMEM((IDXG, CHUNK), jnp.int32),
            pltpu.VMEM((2, CHUNK, N_FEATS), jnp.float32),
            pltpu.VMEM_SHARED((N_NODES, N_FEATS), jnp.float32),
            pltpu.SemaphoreType.DMA,
            pltpu.SemaphoreType.DMA,
        ],
        compiler_params=_SC_PARAMS,
    )(h, src2, dst2, zeros)


def _finalize_body(p_ref, n2_ref, b_ref, out_ref):
    out_ref[...] = (p_ref[0] + p_ref[1]) * n2_ref[...] + b_ref[...]


def _finalize(partials, norm_dst, bias):
    return pl.pallas_call(
        _finalize_body,
        grid=(N_NODES // _MM_BLK,),
        in_specs=[
            pl.BlockSpec((NC, _MM_BLK, N_FEATS), lambda i: (0, i, 0)),
            pl.BlockSpec((_MM_BLK, 1), lambda i: (i, 0)),
            pl.BlockSpec((1, N_FEATS), lambda i: (0, 0)),
        ],
        out_specs=pl.BlockSpec((_MM_BLK, N_FEATS), lambda i: (i, 0)),
        out_shape=jax.ShapeDtypeStruct((N_NODES, N_FEATS), jnp.float32),
    )(partials, norm_dst, bias)


def kernel(features, edge_index, weight, bias):
    ei = edge_index.astype(jnp.int32)
    src = ei[0]
    dst = ei[1]
    hist_src, hist_dst = _degrees(src, dst)
    norm_src, norm_dst = _norms(hist_src, hist_dst)
    h = _matmul(features, norm_src, weight)
    src2 = src.reshape(NW * CHUNKS_PER_TILE, CHUNK)
    dst2 = dst.reshape(NW * CHUNKS_PER_TILE, CHUNK)
    zeros = jnp.zeros((CP_ROWS, N_FEATS), jnp.float32)
    partials = _message(h, src2, dst2, zeros)
    return _finalize(partials, norm_dst, bias.reshape(1, N_FEATS))


# speedup vs baseline: 10.9584x; 1.2043x over previous
import functools

import jax
import jax.numpy as jnp
from jax import lax
from jax.experimental import pallas as pl
from jax.experimental.pallas import tpu as pltpu
from jax.experimental.pallas import tpu_sc as plsc

N_NODES = 10000
N_FEATS = 128
N_EDGES = 320000

NC = 2
NS = 16
NW = NC * NS
EDGES_PER_TILE = N_EDGES // NW
CHUNK = 50
CHUNKS_PER_TILE = EDGES_PER_TILE // CHUNK
IDXG = 40
N_GROUPS = CHUNKS_PER_TILE // IDXG
INNER = 8
NBUF = 4
LOOKAHEAD = 3
LANES = 16
CP_ROWS = 200
CP_CHUNKS = N_NODES // CP_ROWS
CP_ROUNDS = -(-CP_CHUNKS // NS)

_MESH = plsc.VectorSubcoreMesh(
    core_axis_name="c", subcore_axis_name="s", num_cores=NC, num_subcores=NS)
_SC_PARAMS = pltpu.CompilerParams(needs_layout_passes=False)


def _degrees_body(edges_hbm, out_src, out_dst, idx_v, hist_v):
    c = lax.axis_index("c")
    s = lax.axis_index("s")
    wid = c * NS + s
    base = wid * EDGES_PER_TILE
    ones = jnp.full((LANES,), 1.0, jnp.float32)
    zeros = jnp.zeros((LANES,), jnp.float32)

    def one_endpoint(ep_base, out_hbm):
        pltpu.sync_copy(edges_hbm.at[pl.ds(ep_base + base, EDGES_PER_TILE)], idx_v)

        def zero_step(i, _):
            hist_v[pl.ds(i * LANES, LANES)] = zeros
            return 0

        lax.fori_loop(0, N_NODES // LANES, zero_step, 0)

        def add_step(i, _):
            idx = idx_v[pl.ds(i * LANES, LANES)]
            plsc.addupdate_scatter(hist_v, [idx], ones)
            return 0

        lax.fori_loop(0, EDGES_PER_TILE // LANES, add_step, 0)
        pltpu.sync_copy(hist_v, out_hbm.at[wid])

    one_endpoint(0, out_src)
    one_endpoint(N_EDGES, out_dst)


def _degrees(edges_flat):
    return pl.kernel(
        _degrees_body,
        out_type=(
            jax.ShapeDtypeStruct((NW, N_NODES), jnp.float32),
            jax.ShapeDtypeStruct((NW, N_NODES), jnp.float32),
        ),
        mesh=_MESH,
        scratch_types=[
            pltpu.VMEM((EDGES_PER_TILE,), jnp.int32),
            pltpu.VMEM((N_NODES,), jnp.float32),
        ],
        compiler_params=_SC_PARAMS,
    )(edges_flat)


def _norms_scale_body(hs_ref, hd_ref, h0_ref, h_ref, nd_ref):
    ds = jnp.sum(hs_ref[...], axis=0)
    dd = jnp.sum(hd_ref[...], axis=0)
    n1 = lax.rsqrt(jnp.maximum(ds, 1.0))
    h_ref[...] = h0_ref[...] * n1[:, None]
    nd_ref[...] = lax.rsqrt(jnp.maximum(dd, 1.0))[:, None]


def _norms_scale(hist_src, hist_dst, h0):
    return pl.pallas_call(
        _norms_scale_body,
        out_shape=(
            jax.ShapeDtypeStruct((N_NODES, N_FEATS), jnp.float32),
            jax.ShapeDtypeStruct((N_NODES, 1), jnp.float32),
        ),
    )(hist_src, hist_dst, h0)


_MM_BLK = 1000


def _matmul_body(x_ref, w_ref, h_ref):
    h_ref[...] = jnp.dot(x_ref[...], w_ref[...],
                         preferred_element_type=jnp.float32)


def _matmul(features, weight):
    return pl.pallas_call(
        _matmul_body,
        grid=(N_NODES // _MM_BLK,),
        in_specs=[
            pl.BlockSpec((_MM_BLK, N_FEATS), lambda i: (i, 0)),
            pl.BlockSpec((N_FEATS, N_FEATS), lambda i: (0, 0)),
        ],
        out_specs=pl.BlockSpec((_MM_BLK, N_FEATS), lambda i: (i, 0)),
        out_shape=jax.ShapeDtypeStruct((N_NODES, N_FEATS), jnp.float32),
    )(features, weight)


def _message_body(h_hbm, e3d_hbm, zeros_hbm, out_hbm,
                  srcblk, dstblk, rows, acc_sh,
                  gsem0, gsem1, gsem2, gsem3, ssem0, ssem1, ssem2, ssem3):
    c = lax.axis_index("c")
    s = lax.axis_index("s")
    wid = c * NS + s

    for j in range(CP_ROUNDS):
        cid = s + NS * j

        @pl.when(cid < CP_CHUNKS)
        def _():
            pltpu.sync_copy(zeros_hbm, acc_sh.at[pl.ds(cid * CP_ROWS, CP_ROWS)])

    plsc.subcore_barrier()

    gsems = (gsem0, gsem1, gsem2, gsem3)
    ssems = (ssem0, ssem1, ssem2, ssem3)

    def gather(j, b):
        return pltpu.async_copy(h_hbm.at[srcblk.at[j]], rows.at[b], gsems[b])

    def wait_gather(j, b):
        pltpu.make_async_copy(h_hbm.at[srcblk.at[j]], rows.at[b],
                              gsems[b]).wait()

    def scatter(j, b):
        return pltpu.async_copy(rows.at[b], acc_sh.at[dstblk.at[j]], ssems[b],
                                add=True)

    def wait_scatter(j, b):
        pltpu.make_async_copy(rows.at[b], acc_sh.at[dstblk.at[j]],
                              ssems[b]).wait()

    def group_step(g, _):
        base = wid * CHUNKS_PER_TILE + g * IDXG
        pltpu.sync_copy(e3d_hbm.at[0, pl.ds(base, IDXG)], srcblk)
        pltpu.sync_copy(e3d_hbm.at[1, pl.ds(base, IDXG)], dstblk)
        for p in range(LOOKAHEAD):
            gather(p, p % NBUF)

        def inner_step(q, _):
            for k in range(INNER):
                j = q * INNER + k
                jn = j + LOOKAHEAD
                bn = (k + LOOKAHEAD) % NBUF

                @pl.when(jn < IDXG)
                def _():
                    @pl.when(jn >= NBUF)
                    def _():
                        wait_scatter(jn - NBUF, bn)

                    gather(jn, bn)

                wait_gather(j, k % NBUF)
                scatter(j, k % NBUF)
            return 0

        lax.fori_loop(0, IDXG // INNER, inner_step, 0)
        for j in range(IDXG - NBUF, IDXG):
            wait_scatter(j, j % NBUF)
        return 0

    lax.fori_loop(0, N_GROUPS, group_step, 0)
    plsc.subcore_barrier()

    for j in range(CP_ROUNDS):
        cid = s + NS * j

        @pl.when(cid < CP_CHUNKS)
        def _():
            pltpu.sync_copy(acc_sh.at[pl.ds(cid * CP_ROWS, CP_ROWS)],
                            out_hbm.at[c, pl.ds(cid * CP_ROWS, CP_ROWS)])


def _message(h, e3d, zeros):
    return pl.kernel(
        _message_body,
        out_type=jax.ShapeDtypeStruct((NC, N_NODES, N_FEATS), jnp.float32),
        mesh=_MESH,
        scratch_types=[
            pltpu.VMEM((IDXG, CHUNK), jnp.int32),
            pltpu.VMEM((IDXG, CHUNK), jnp.int32),
            pltpu.VMEM((NBUF, CHUNK, N_FEATS), jnp.float32),
            pltpu.VMEM_SHARED((N_NODES, N_FEATS), jnp.float32),
            pltpu.SemaphoreType.DMA,
            pltpu.SemaphoreType.DMA,
            pltpu.SemaphoreType.DMA,
            pltpu.SemaphoreType.DMA,
            pltpu.SemaphoreType.DMA,
            pltpu.SemaphoreType.DMA,
            pltpu.SemaphoreType.DMA,
            pltpu.SemaphoreType.DMA,
        ],
        compiler_params=_SC_PARAMS,
    )(h, e3d, zeros)


def _finalize_body(p_ref, n2_ref, b_ref, out_ref):
    out_ref[...] = (p_ref[0] + p_ref[1]) * n2_ref[...] + b_ref[...]


def _finalize(partials, norm_dst, bias):
    return pl.pallas_call(
        _finalize_body,
        grid=(N_NODES // _MM_BLK,),
        in_specs=[
            pl.BlockSpec((NC, _MM_BLK, N_FEATS), lambda i: (0, i, 0)),
            pl.BlockSpec((_MM_BLK, 1), lambda i: (i, 0)),
            pl.BlockSpec((1, N_FEATS), lambda i: (0, 0)),
        ],
        out_specs=pl.BlockSpec((_MM_BLK, N_FEATS), lambda i: (i, 0)),
        out_shape=jax.ShapeDtypeStruct((N_NODES, N_FEATS), jnp.float32),
    )(partials, norm_dst, bias)


def kernel(features, edge_index, weight, bias):
    ei = edge_index.astype(jnp.int32)
    edges_flat = ei.reshape(2 * N_EDGES)
    e3d = ei.reshape(2, NW * CHUNKS_PER_TILE, CHUNK)
    hist_src, hist_dst = _degrees(edges_flat)
    h0 = _matmul(features, weight)
    h, norm_dst = _norms_scale(hist_src, hist_dst, h0)
    zeros = jnp.zeros((CP_ROWS, N_FEATS), jnp.float32)
    partials = _message(h, e3d, zeros)
    return _finalize(partials, norm_dst, bias.reshape(1, N_FEATS))
